# Initial kernel scaffold; baseline (speedup 1.0000x reference)
import jax
import jax.numpy as jnp
from jax.experimental import pallas as pl


def kernel(tkn_ids, table, pos_encoding):
    raise NotImplementedError("write your pallas kernel here")


# speedup vs baseline: 1.0636x; 1.0636x over previous
import functools

import jax
import jax.numpy as jnp
from jax import lax
from jax.experimental import pallas as pl
from jax.experimental.pallas import tpu as pltpu
from jax.experimental.pallas import tpu_sc as plsc

VOCAB_SIZE = 100000
EMBED_DIM = 128
MAX_SEQ = 2048
BATCH = 4

_B = BATCH * MAX_SEQ
_INFO = plsc.get_sparse_core_info()
_NC = _INFO.num_cores
_NS = _INFO.num_subcores
_NW = _NC * _NS
_BPW = _B // _NW
_CHUNK = 128
_NCHUNK = _BPW // _CHUNK


@functools.partial(
    pl.kernel,
    mesh=plsc.VectorSubcoreMesh(core_axis_name="c", subcore_axis_name="s"),
    out_type=jax.ShapeDtypeStruct((_B, EMBED_DIM), jnp.float32),
    scratch_types=[
        pltpu.VMEM((_NCHUNK, _CHUNK), jnp.int32),
        pltpu.VMEM((_BPW, EMBED_DIM), jnp.float32),
        pltpu.VMEM((_BPW, EMBED_DIM), jnp.float32),
        pltpu.SemaphoreType.DMA,
        pltpu.SemaphoreType.DMA,
    ],
)
def _sc_embed(idx_hbm, table_hbm, pos_hbm, out_hbm,
              idx_v, rows_v, pos_v, gsem, psem):
    wid = lax.axis_index("s") * _NC + lax.axis_index("c")
    base = wid * _BPW
    pos_base = lax.rem(base, MAX_SEQ)

    pltpu.sync_copy(idx_hbm.at[pl.ds(wid * _NCHUNK, _NCHUNK)], idx_v)

    pos_cp = pltpu.async_copy(pos_hbm.at[pl.ds(pos_base, _BPW)], pos_v, psem)
    gathers = []
    for j in range(_NCHUNK):
        gathers.append(pltpu.async_copy(
            table_hbm.at[idx_v.at[j]],
            rows_v.at[pl.ds(j * _CHUNK, _CHUNK)],
            gsem))
    pos_cp.wait()
    for cp in gathers:
        cp.wait()

    def add_row(i, carry):
        for j in range(EMBED_DIM // 16):
            sl = pl.ds(j * 16, 16)
            rows_v[i, sl] = rows_v[i, sl] + pos_v[i, sl]
        return carry
    lax.fori_loop(0, _BPW, add_row, 0)

    pltpu.sync_copy(rows_v, out_hbm.at[pl.ds(base, _BPW)])


def kernel(tkn_ids, table, pos_encoding):
    idx = tkn_ids.astype(jnp.int32).reshape(_NW * _NCHUNK, _CHUNK)
    pos = pos_encoding.reshape(MAX_SEQ, EMBED_DIM).astype(jnp.float32)
    out = _sc_embed(idx, table, pos)
    return out.reshape(BATCH, MAX_SEQ, EMBED_DIM)


# speedup vs baseline: 1.0787x; 1.0142x over previous
import functools

import jax
import jax.numpy as jnp
from jax import lax
from jax.experimental import pallas as pl
from jax.experimental.pallas import tpu as pltpu
from jax.experimental.pallas import tpu_sc as plsc

VOCAB_SIZE = 100000
EMBED_DIM = 128
MAX_SEQ = 2048
BATCH = 4

_B = BATCH * MAX_SEQ
_INFO = plsc.get_sparse_core_info()
_NC = _INFO.num_cores
_NS = _INFO.num_subcores
_NW = _NC * _NS
_BPW = _B // _NW
_CHUNK = 64
_NCHUNK = _BPW // _CHUNK


@functools.partial(
    pl.kernel,
    mesh=plsc.VectorSubcoreMesh(core_axis_name="c", subcore_axis_name="s"),
    out_type=jax.ShapeDtypeStruct((_B, EMBED_DIM), jnp.float32),
    scratch_types=[
        pltpu.VMEM((_NCHUNK, _CHUNK), jnp.int32),
        pltpu.VMEM((_BPW, EMBED_DIM), jnp.float32),
        pltpu.VMEM((_BPW, EMBED_DIM), jnp.float32),
        pltpu.SemaphoreType.DMA,
        pltpu.SemaphoreType.DMA,
    ]
    + [pltpu.SemaphoreType.DMA for _ in range(_NCHUNK)],
)
def _sc_embed(idx_hbm, table_hbm, pos_hbm, out_hbm,
              idx_v, rows_v, pos_v, psem, wsem, *gsems):
    wid = lax.axis_index("s") * _NC + lax.axis_index("c")
    base = wid * _BPW
    pos_base = lax.rem(base, MAX_SEQ)

    pltpu.sync_copy(idx_hbm.at[pl.ds(wid * _NCHUNK, _NCHUNK)], idx_v)

    pos_cp = pltpu.async_copy(pos_hbm.at[pl.ds(pos_base, _BPW)], pos_v, psem)
    gathers = []
    for j in range(_NCHUNK):
        gathers.append(pltpu.async_copy(
            table_hbm.at[idx_v.at[j]],
            rows_v.at[pl.ds(j * _CHUNK, _CHUNK)],
            gsems[j]))
    pos_cp.wait()

    writes = []
    for j in range(_NCHUNK):
        gathers[j].wait()
        lo = j * _CHUNK

        def add_row(i, carry):
            for k in range(EMBED_DIM // 16):
                sl = pl.ds(k * 16, 16)
                rows_v[i, sl] = rows_v[i, sl] + pos_v[i, sl]
            return carry
        lax.fori_loop(lo, lo + _CHUNK, add_row, 0)
        writes.append(pltpu.async_copy(
            rows_v.at[pl.ds(lo, _CHUNK)],
            out_hbm.at[pl.ds(base + lo, _CHUNK)],
            wsem))
    for cp in writes:
        cp.wait()


def kernel(tkn_ids, table, pos_encoding):
    idx = tkn_ids.astype(jnp.int32).reshape(_NW * _NCHUNK, _CHUNK)
    pos = pos_encoding.reshape(MAX_SEQ, EMBED_DIM).astype(jnp.float32)
    out = _sc_embed(idx, table, pos)
    return out.reshape(BATCH, MAX_SEQ, EMBED_DIM)


# speedup vs baseline: 1.1545x; 1.0703x over previous
import functools

import jax
import jax.numpy as jnp
from jax import lax
from jax.experimental import pallas as pl
from jax.experimental.pallas import tpu as pltpu
from jax.experimental.pallas import tpu_sc as plsc

VOCAB_SIZE = 100000
EMBED_DIM = 128
MAX_SEQ = 2048
BATCH = 4

_B = BATCH * MAX_SEQ
_INFO = plsc.get_sparse_core_info()
_NC = _INFO.num_cores
_NS = _INFO.num_subcores
_NW = _NC * _NS
_SPT = MAX_SEQ // _NW


@functools.partial(
    pl.kernel,
    mesh=plsc.VectorSubcoreMesh(core_axis_name="c", subcore_axis_name="s"),
    out_type=jax.ShapeDtypeStruct((_B, EMBED_DIM), jnp.float32),
    scratch_types=[
        pltpu.VMEM((BATCH, _SPT), jnp.int32),
        pltpu.VMEM((BATCH * _SPT, EMBED_DIM), jnp.float32),
        pltpu.VMEM((_SPT, EMBED_DIM), jnp.float32),
        pltpu.SemaphoreType.DMA,
        pltpu.SemaphoreType.DMA,
    ]
    + [pltpu.SemaphoreType.DMA for _ in range(BATCH)],
)
def _sc_embed(idx_hbm, table_hbm, pos_hbm, out_hbm,
              idx_v, rows_v, pos_v, psem, wsem, *gsems):
    wid = lax.axis_index("s") * _NC + lax.axis_index("c")
    col0 = wid * _SPT

    idx_cps = [
        pltpu.async_copy(idx_hbm.at[b, pl.ds(col0, _SPT)], idx_v.at[b], psem)
        for b in range(BATCH)
    ]
    for cp in idx_cps:
        cp.wait()

    pos_cp = pltpu.async_copy(pos_hbm.at[pl.ds(col0, _SPT)], pos_v, psem)
    gathers = []
    for b in range(BATCH):
        gathers.append(pltpu.async_copy(
            table_hbm.at[idx_v.at[b]],
            rows_v.at[pl.ds(b * _SPT, _SPT)],
            gsems[b]))
    pos_cp.wait()

    writes = []
    for b in range(BATCH):
        gathers[b].wait()
        lo = b * _SPT

        def add_row(i, carry):
            for k in range(EMBED_DIM // 16):
                sl = pl.ds(k * 16, 16)
                rows_v[lo + i, sl] = rows_v[lo + i, sl] + pos_v[i, sl]
            return carry
        lax.fori_loop(0, _SPT, add_row, 0)
        writes.append(pltpu.async_copy(
            rows_v.at[pl.ds(lo, _SPT)],
            out_hbm.at[pl.ds(b * MAX_SEQ + col0, _SPT)],
            wsem))
    for cp in writes:
        cp.wait()


def kernel(tkn_ids, table, pos_encoding):
    idx = tkn_ids.astype(jnp.int32)
    pos = pos_encoding.reshape(MAX_SEQ, EMBED_DIM).astype(jnp.float32)
    out = _sc_embed(idx, table, pos)
    return out.reshape(BATCH, MAX_SEQ, EMBED_DIM)
